# baseline (speedup 1.0000x reference)
import functools

import jax
import jax.numpy as jnp
from jax import lax
from jax.experimental import pallas as pl
from jax.experimental.pallas import tpu as pltpu
from jax.experimental.pallas import tpu_sc as plsc

N = 10000
E = 160000
D = 256
H = 8
HD = D // H
ALPHA = 0.2

NC = 2
NS = 16
HC = H // NC

EPT = E // NS
EPTP = 10080
EP = NS * EPTP
C = 96
NCHUNK = EPTP // C
ROWS_PT = 624
TAILOFF = NS * ROWS_PT
TAIL = N - TAILOFF
DLEN = 2560
DPAD = NS * DLEN
SW = 16
NSP = (HC * HD) // SW


def _proj_body(x_ref, w_ref, as_ref, ad_ref, ha_ref, hb_ref, s_ref, d_ref):
    x = x_ref[...]
    blk = x.shape[0]
    hs, ss, ds = [], [], []
    for h in range(H):
        hh = jnp.dot(x, w_ref[h], preferred_element_type=jnp.float32)
        hs.append(hh)
        ss.append(jnp.dot(hh, as_ref[h], preferred_element_type=jnp.float32))
        ds.append(jnp.dot(hh, ad_ref[h], preferred_element_type=jnp.float32))
    ha_ref[...] = jnp.concatenate(hs[:HC], axis=1)
    hb_ref[...] = jnp.concatenate(hs[HC:], axis=1)
    pad = jnp.zeros((blk, 16 - H), jnp.float32)
    s_ref[...] = jnp.concatenate(ss + [pad], axis=1)
    d_ref[...] = jnp.concatenate(ds + [pad], axis=1)


def _infl_body(ew_ref, pw_ref, pb_ref, o_ref):
    vals = ew_ref[...] * pw_ref[...] + pb_ref[...]
    pad = jnp.full((EPTP - EPT, 16), -1e30, jnp.float32)
    o_ref[...] = jnp.concatenate([vals, pad], axis=0)


def _sc_body(hA, hB, asp, adp, inflP, srcI, dstI, zrows, zflat,
             outA, outB,
             srcp, dstp, arows_s, arows_d, inflc, eidx, idxg, idxb,
             gbuf, sbuf, zhm, invf, acc_sp, den_sp, gsem, ssem):
    c = lax.axis_index("c")
    s = lax.axis_index("s")
    iota = lax.iota(jnp.int32, 16)
    zero16 = jnp.zeros((16,), jnp.int32)

    def _zero_acc():
        pltpu.sync_copy(zrows, acc_sp.at[pl.ds(s * ROWS_PT, ROWS_PT)])

        @pl.when(s == 0)
        def _zt():
            pltpu.sync_copy(zrows.at[pl.ds(0, TAIL)],
                            acc_sp.at[pl.ds(TAILOFF, TAIL)])

    _zero_acc()
    pltpu.sync_copy(zflat, den_sp.at[pl.ds(s * DLEN, DLEN)])
    pltpu.sync_copy(srcI.at[pl.ds(s * EPTP, EPTP)], srcp)
    pltpu.sync_copy(dstI.at[pl.ds(s * EPTP, EPTP)], dstp)
    plsc.subcore_barrier()

    hmask = (iota >= c * HC) & (iota < c * HC + HC)
    lane = iota - c * HC

    def _fire_a(k):
        slot = k % 2
        pltpu.async_copy(asp.at[srcp.at[pl.ds(k * C, C)]],
                         arows_s.at[slot], gsem.at[slot])
        pltpu.async_copy(adp.at[dstp.at[pl.ds(k * C, C)]],
                         arows_d.at[slot], gsem.at[slot])
        pltpu.async_copy(inflP.at[pl.ds(s * EPTP + k * C, C)],
                         inflc.at[slot], gsem.at[slot])

    def _wait_a(slot):
        pltpu.make_async_copy(asp.at[srcp.at[pl.ds(0, C)]],
                              arows_s.at[slot], gsem.at[slot]).wait()
        pltpu.make_async_copy(adp.at[dstp.at[pl.ds(0, C)]],
                              arows_d.at[slot], gsem.at[slot]).wait()
        pltpu.make_async_copy(inflP.at[pl.ds(0, C)],
                              inflc.at[slot], gsem.at[slot]).wait()

    def _wait_a_scat(slot):
        for h in range(HC):
            pltpu.make_async_copy(
                zhm.at[pl.ds(h * C, C)],
                den_sp.at[eidx.at[slot * HC + h]], ssem.at[slot]).wait()

    def _chunk_a(k, carry):
        slot = k % 2
        _wait_a(slot)
        for j in range(C // 16):
            dstv = dstp[pl.ds(k * C + j * 16, 16)]
            for h in range(HC):
                eidx[slot * HC + h, pl.ds(j * 16, 16)] = dstv * HC + h
        kc = k * C

        @plsc.parallel_loop(0, C, unroll=8)
        def edge_a(e):
            t = arows_s[slot, e] + arows_d[slot, e]
            t = jnp.where(t >= 0.0, t, t * ALPHA)
            t = t + inflc[slot, e]
            z = jnp.exp(t)
            plsc.store_scatter(zhm, [lane * EPTP + (kc + e)], z, mask=hmask)

        @pl.when(k >= 2)
        def _():
            _wait_a_scat(slot)

        for h in range(HC):
            pltpu.async_copy(zhm.at[pl.ds(h * EPTP + kc, C)],
                             den_sp.at[eidx.at[slot * HC + h]], ssem.at[slot],
                             add=True)

        @pl.when(k < NCHUNK - 2)
        def _():
            _fire_a(k + 2)
        return carry

    def _prol_a(k, carry):
        _fire_a(k)
        return carry

    lax.fori_loop(0, 2, _prol_a, 0)
    lax.fori_loop(0, NCHUNK, _chunk_a, 0)
    _wait_a_scat(0)
    _wait_a_scat(1)
    plsc.subcore_barrier()

    off = s * DLEN
    pltpu.sync_copy(den_sp.at[pl.ds(off, DLEN)], invf.at[pl.ds(0, DLEN)])

    def recip(i, carry):
        v = invf[pl.ds(i * 16, 16)]
        invf[pl.ds(i * 16, 16)] = 1.0 / (v + 1e-16)
        return carry

    lax.fori_loop(0, DLEN // 16, recip, 0)
    pltpu.sync_copy(invf.at[pl.ds(0, DLEN)], den_sp.at[pl.ds(off, DLEN)])
    plsc.subcore_barrier()
    pltpu.sync_copy(den_sp, invf)

    def wtrans(i, carry):
        dstv = dstp[pl.ds(i * 16, 16)]
        for p in range(HC):
            o3 = p * EPTP + i * 16
            zhm[pl.ds(o3, 16)] = (zhm[pl.ds(o3, 16)] *
                                  plsc.load_gather(invf, [dstv * HC + p]))
        return carry

    lax.fori_loop(0, EPTP // 16, wtrans, 0)

    def _fire_b(k, p):
        slot = k % 2
        base = k * C
        for j in range(C // 16):
            idxg[slot, pl.ds(j * 16, 16)] = (
                srcp[pl.ds(base + j * 16, 16)] * NSP + p)

        @pl.when(c == 0)
        def _():
            pltpu.async_copy(hA.at[idxg.at[slot]], gbuf.at[slot],
                             gsem.at[slot])

        @pl.when(c == 1)
        def _():
            pltpu.async_copy(hB.at[idxg.at[slot]], gbuf.at[slot],
                             gsem.at[slot])

    def _chunk_b(k, p):
        slot = k % 2
        pltpu.make_async_copy(hA.at[idxg.at[slot]], gbuf.at[slot],
                              gsem.at[slot]).wait()
        for j in range(C // 16):
            idxb[slot, pl.ds(j * 16, 16)] = dstp[pl.ds(k * C + j * 16, 16)]

        @pl.when(k >= 2)
        def _():
            pltpu.make_async_copy(sbuf.at[slot],
                                  acc_sp.at[idxb.at[slot]],
                                  ssem.at[slot]).wait()

        woff = (p // (HD // SW)) * EPTP + k * C

        @plsc.parallel_loop(0, C // 16, unroll=2)
        def grp_b(j):
            wv = zhm[pl.ds(woff + j * 16, 16)]
            for l in range(16):
                ee = j * 16 + l
                for q in range(SW // 16):
                    sbuf[slot, ee, pl.ds(q * 16, 16)] = (
                        gbuf[slot, ee, pl.ds(q * 16, 16)] * wv[l])
        pltpu.async_copy(sbuf.at[slot], acc_sp.at[idxb.at[slot]],
                         ssem.at[slot], add=True)

        @pl.when(k < NCHUNK - 2)
        def _():
            _fire_b(k + 2, p)

    def subpass(p, carry):
        def _prol_b(k, carry2):
            _fire_b(k, p)
            return carry2

        lax.fori_loop(0, 2, _prol_b, 0)

        def loop_b(k, carry2):
            _chunk_b(k, p)
            return carry2

        lax.fori_loop(0, NCHUNK, loop_b, 0)
        for slot in range(2):
            pltpu.make_async_copy(sbuf.at[slot], acc_sp.at[idxb.at[slot]],
                                  ssem.at[slot]).wait()
        plsc.subcore_barrier()

        @pl.when(c == 0)
        def _o0():
            pltpu.sync_copy(acc_sp.at[pl.ds(s * ROWS_PT, ROWS_PT)],
                            outA.at[p, pl.ds(s * ROWS_PT, ROWS_PT)])

            @pl.when(s == 0)
            def _ot0():
                pltpu.sync_copy(acc_sp.at[pl.ds(TAILOFF, TAIL)],
                                outA.at[p, pl.ds(TAILOFF, TAIL)])

        @pl.when(c == 1)
        def _o1():
            pltpu.sync_copy(acc_sp.at[pl.ds(s * ROWS_PT, ROWS_PT)],
                            outB.at[p, pl.ds(s * ROWS_PT, ROWS_PT)])

            @pl.when(s == 0)
            def _ot1():
                pltpu.sync_copy(acc_sp.at[pl.ds(TAILOFF, TAIL)],
                                outB.at[p, pl.ds(TAILOFF, TAIL)])

        plsc.subcore_barrier()
        _zero_acc()
        plsc.subcore_barrier()
        return carry

    lax.fori_loop(0, NSP, subpass, 0)


_sc_call = functools.partial(
    pl.kernel,
    _sc_body,
    out_type=(jax.ShapeDtypeStruct((NSP, N, SW), jnp.float32),) * 2,
    mesh=plsc.VectorSubcoreMesh(core_axis_name="c", subcore_axis_name="s"),
    compiler_params=pltpu.CompilerParams(use_tc_tiling_on_sc=False,
                                         needs_layout_passes=False),
    scratch_types=(
        pltpu.VMEM((EPTP,), jnp.int32),
        pltpu.VMEM((EPTP,), jnp.int32),
        pltpu.VMEM((2, C, 16), jnp.float32),
        pltpu.VMEM((2, C, 16), jnp.float32),
        pltpu.VMEM((2, C, 16), jnp.float32),
        pltpu.VMEM((2 * HC, C), jnp.int32),
        pltpu.VMEM((2, C), jnp.int32),
        pltpu.VMEM((2, C), jnp.int32),
        pltpu.VMEM((2, C, SW), jnp.float32),
        pltpu.VMEM((2, C, SW), jnp.float32),
        pltpu.VMEM((HC * EPTP,), jnp.float32),
        pltpu.VMEM((DPAD,), jnp.float32),
        pltpu.VMEM_SHARED((N, SW), jnp.float32),
        pltpu.VMEM_SHARED((DPAD,), jnp.float32),
        pltpu.SemaphoreType.DMA((2,)),
        pltpu.SemaphoreType.DMA((2,)),
    ),
)()


def kernel(x, edge_index, edge_weight, W, a_src, a_dst, edge_proj_w,
           edge_proj_b, bias):
    src = edge_index[0]
    dst = edge_index[1]

    BLK = 1000
    hA, hB, asp, adp = pl.pallas_call(
        _proj_body,
        grid=(N // BLK,),
        in_specs=[
            pl.BlockSpec((BLK, D), lambda i: (i, 0)),
            pl.BlockSpec((H, D, HD), lambda i: (0, 0, 0)),
            pl.BlockSpec((H, HD, 1), lambda i: (0, 0, 0)),
            pl.BlockSpec((H, HD, 1), lambda i: (0, 0, 0)),
        ],
        out_specs=[
            pl.BlockSpec((BLK, HC * HD), lambda i: (i, 0)),
            pl.BlockSpec((BLK, HC * HD), lambda i: (i, 0)),
            pl.BlockSpec((BLK, 16), lambda i: (i, 0)),
            pl.BlockSpec((BLK, 16), lambda i: (i, 0)),
        ],
        out_shape=[
            jax.ShapeDtypeStruct((N, HC * HD), jnp.float32),
            jax.ShapeDtypeStruct((N, HC * HD), jnp.float32),
            jax.ShapeDtypeStruct((N, 16), jnp.float32),
            jax.ShapeDtypeStruct((N, 16), jnp.float32),
        ],
    )(x, W, a_src, a_dst)

    pw_p = jnp.concatenate([edge_proj_w.reshape(1, H),
                            jnp.zeros((1, 16 - H), jnp.float32)], axis=1)
    pb_p = jnp.concatenate([edge_proj_b.reshape(1, H),
                            jnp.zeros((1, 16 - H), jnp.float32)], axis=1)
    infl_p = pl.pallas_call(
        _infl_body,
        grid=(NS,),
        in_specs=[
            pl.BlockSpec((EPT, 1), lambda i: (i, 0)),
            pl.BlockSpec((1, 16), lambda i: (0, 0)),
            pl.BlockSpec((1, 16), lambda i: (0, 0)),
        ],
        out_specs=pl.BlockSpec((EPTP, 16), lambda i: (i, 0)),
        out_shape=jax.ShapeDtypeStruct((EP, 16), jnp.float32),
    )(edge_weight.reshape(E, 1), pw_p, pb_p)

    padn = EPTP - EPT
    src_p = jnp.concatenate(
        [src.reshape(NS, EPT), jnp.zeros((NS, padn), jnp.int32)],
        axis=1).reshape(-1)
    dst_p = jnp.concatenate(
        [dst.reshape(NS, EPT), jnp.zeros((NS, padn), jnp.int32)],
        axis=1).reshape(-1)

    hA = hA.reshape(N * NSP, SW)
    hB = hB.reshape(N * NSP, SW)
    zrows = jnp.zeros((ROWS_PT, SW), jnp.float32)
    zflat = jnp.zeros((DLEN,), jnp.float32)
    oA, oB = _sc_call(hA, hB, asp, adp, infl_p, src_p, dst_p, zrows, zflat)
    out = jnp.concatenate(
        [oA.transpose(1, 0, 2).reshape(N, HC * HD),
         oB.transpose(1, 0, 2).reshape(N, HC * HD)], axis=1)
    return out + bias


# speedup vs baseline: 1.0372x; 1.0372x over previous
import functools

import jax
import jax.numpy as jnp
from jax import lax
from jax.experimental import pallas as pl
from jax.experimental.pallas import tpu as pltpu
from jax.experimental.pallas import tpu_sc as plsc

N = 10000
E = 160000
D = 256
H = 8
HD = D // H
ALPHA = 0.2

NC = 2
NS = 16
HC = H // NC

EPT = E // NS
EPTP = 10080
EP = NS * EPTP
CA = 80
NCHA = EPTP // CA
CB = 112
NCHB = EPTP // CB
ROWS_PT = 624
TAILOFF = NS * ROWS_PT
TAIL = N - TAILOFF
DLEN = 2560
DPAD = NS * DLEN
SW = 16
NSP = (HC * HD) // SW


def _proj_body(x_ref, w_ref, as_ref, ad_ref, ha_ref, hb_ref, s_ref, d_ref):
    h = jnp.dot(x_ref[...], w_ref[...], preferred_element_type=jnp.float32)
    blk = h.shape[0]
    ha_ref[...] = h[:, :HC * HD]
    hb_ref[...] = h[:, HC * HD:]
    pad = jnp.zeros((blk, H), jnp.float32)
    s = jnp.dot(h, as_ref[...], preferred_element_type=jnp.float32)
    d = jnp.dot(h, ad_ref[...], preferred_element_type=jnp.float32)
    s_ref[...] = jnp.concatenate([s, pad], axis=1)
    d_ref[...] = jnp.concatenate([d, pad], axis=1)


def _infl_body(ew_ref, pw_ref, pb_ref, o_ref):
    vals = ew_ref[...] * pw_ref[...] + pb_ref[...]
    pad = jnp.full((EPTP - EPT, 16), -1e30, jnp.float32)
    o_ref[...] = jnp.concatenate([vals, pad], axis=0)


def _sc_body(hA, hB, asp, adp, inflP, srcI, dstI, zrows, zflat,
             outA, outB,
             srcp, dstp, arows_s, arows_d, inflc, eidx, idxg, idxb,
             gbuf, sbuf, zhm, invf, acc_sp, den_sp, gsem, ssem):
    c = lax.axis_index("c")
    s = lax.axis_index("s")
    iota = lax.iota(jnp.int32, 16)
    zero16 = jnp.zeros((16,), jnp.int32)

    def _zero_acc():
        pltpu.sync_copy(zrows, acc_sp.at[pl.ds(s * ROWS_PT, ROWS_PT)])

        @pl.when(s == 0)
        def _zt():
            pltpu.sync_copy(zrows.at[pl.ds(0, TAIL)],
                            acc_sp.at[pl.ds(TAILOFF, TAIL)])

    _zero_acc()
    pltpu.sync_copy(zflat, den_sp.at[pl.ds(s * DLEN, DLEN)])
    pltpu.sync_copy(srcI.at[pl.ds(s * EPTP, EPTP)], srcp)
    pltpu.sync_copy(dstI.at[pl.ds(s * EPTP, EPTP)], dstp)
    plsc.subcore_barrier()

    hmask = (iota >= c * HC) & (iota < c * HC + HC)
    lane = iota - c * HC

    def _fire_a(k):
        slot = k % 2
        pltpu.async_copy(asp.at[srcp.at[pl.ds(k * CA, CA)]],
                         arows_s.at[slot], gsem.at[slot])
        pltpu.async_copy(adp.at[dstp.at[pl.ds(k * CA, CA)]],
                         arows_d.at[slot], gsem.at[slot])
        pltpu.async_copy(inflP.at[pl.ds(s * EPTP + k * CA, CA)],
                         inflc.at[slot], gsem.at[slot])

    def _wait_a(slot):
        pltpu.make_async_copy(asp.at[srcp.at[pl.ds(0, CA)]],
                              arows_s.at[slot], gsem.at[slot]).wait()
        pltpu.make_async_copy(adp.at[dstp.at[pl.ds(0, CA)]],
                              arows_d.at[slot], gsem.at[slot]).wait()
        pltpu.make_async_copy(inflP.at[pl.ds(0, CA)],
                              inflc.at[slot], gsem.at[slot]).wait()

    def _wait_a_scat(slot):
        for h in range(HC):
            pltpu.make_async_copy(
                zhm.at[pl.ds(h * CA, CA)],
                den_sp.at[eidx.at[slot * HC + h]], ssem.at[slot]).wait()

    def _chunk_a(k, carry):
        slot = k % 2
        _wait_a(slot)
        for j in range(CA // 16):
            dstv = dstp[pl.ds(k * CA + j * 16, 16)]
            for h in range(HC):
                eidx[slot * HC + h, pl.ds(j * 16, 16)] = dstv * HC + h
        kc = k * CA

        @plsc.parallel_loop(0, CA, unroll=8)
        def edge_a(e):
            t = arows_s[slot, e] + arows_d[slot, e]
            t = jnp.where(t >= 0.0, t, t * ALPHA)
            t = t + inflc[slot, e]
            z = jnp.exp(t)
            plsc.store_scatter(zhm, [lane * EPTP + (kc + e)], z, mask=hmask)

        @pl.when(k >= 2)
        def _():
            _wait_a_scat(slot)

        for h in range(HC):
            pltpu.async_copy(zhm.at[pl.ds(h * EPTP + kc, CA)],
                             den_sp.at[eidx.at[slot * HC + h]], ssem.at[slot],
                             add=True)

        @pl.when(k < NCHA - 2)
        def _():
            _fire_a(k + 2)
        return carry

    def _prol_a(k, carry):
        _fire_a(k)
        return carry

    lax.fori_loop(0, 2, _prol_a, 0)
    lax.fori_loop(0, NCHA, _chunk_a, 0)
    _wait_a_scat(0)
    _wait_a_scat(1)
    plsc.subcore_barrier()

    off = s * DLEN
    pltpu.sync_copy(den_sp.at[pl.ds(off, DLEN)], invf.at[pl.ds(0, DLEN)])

    def recip(i, carry):
        v = invf[pl.ds(i * 16, 16)]
        invf[pl.ds(i * 16, 16)] = 1.0 / (v + 1e-16)
        return carry

    lax.fori_loop(0, DLEN // 16, recip, 0)
    pltpu.sync_copy(invf.at[pl.ds(0, DLEN)], den_sp.at[pl.ds(off, DLEN)])
    plsc.subcore_barrier()
    pltpu.sync_copy(den_sp, invf)

    def wtrans(i, carry):
        dstv = dstp[pl.ds(i * 16, 16)]
        for p in range(HC):
            o3 = p * EPTP + i * 16
            zhm[pl.ds(o3, 16)] = (zhm[pl.ds(o3, 16)] *
                                  plsc.load_gather(invf, [dstv * HC + p]))
        return carry

    lax.fori_loop(0, EPTP // 16, wtrans, 0)

    def _fire_b(k, p):
        slot = k % 2
        base = k * CB
        for j in range(CB // 16):
            idxg[slot, pl.ds(j * 16, 16)] = (
                srcp[pl.ds(base + j * 16, 16)] * NSP + p)

        @pl.when(c == 0)
        def _():
            pltpu.async_copy(hA.at[idxg.at[slot]], gbuf.at[slot],
                             gsem.at[slot])

        @pl.when(c == 1)
        def _():
            pltpu.async_copy(hB.at[idxg.at[slot]], gbuf.at[slot],
                             gsem.at[slot])

    def _chunk_b(k, p):
        slot = k % 2
        pltpu.make_async_copy(hA.at[idxg.at[slot]], gbuf.at[slot],
                              gsem.at[slot]).wait()
        for j in range(CB // 16):
            idxb[slot, pl.ds(j * 16, 16)] = dstp[pl.ds(k * CB + j * 16, 16)]

        @pl.when(k >= 2)
        def _():
            pltpu.make_async_copy(sbuf.at[slot],
                                  acc_sp.at[idxb.at[slot]],
                                  ssem.at[slot]).wait()

        woff = (p // (HD // SW)) * EPTP + k * CB

        @plsc.parallel_loop(0, CB // 16, unroll=2)
        def grp_b(j):
            wv = zhm[pl.ds(woff + j * 16, 16)]
            for l in range(16):
                ee = j * 16 + l
                for q in range(SW // 16):
                    sbuf[slot, ee, pl.ds(q * 16, 16)] = (
                        gbuf[slot, ee, pl.ds(q * 16, 16)] * wv[l])
        pltpu.async_copy(sbuf.at[slot], acc_sp.at[idxb.at[slot]],
                         ssem.at[slot], add=True)

        @pl.when(k < NCHB - 2)
        def _():
            _fire_b(k + 2, p)

    def subpass(p, carry):
        def _prol_b(k, carry2):
            _fire_b(k, p)
            return carry2

        lax.fori_loop(0, 2, _prol_b, 0)

        def loop_b(k, carry2):
            _chunk_b(k, p)
            return carry2

        lax.fori_loop(0, NCHB, loop_b, 0)
        for slot in range(2):
            pltpu.make_async_copy(sbuf.at[slot], acc_sp.at[idxb.at[slot]],
                                  ssem.at[slot]).wait()
        plsc.subcore_barrier()

        @pl.when(c == 0)
        def _o0():
            pltpu.sync_copy(acc_sp.at[pl.ds(s * ROWS_PT, ROWS_PT)],
                            outA.at[p, pl.ds(s * ROWS_PT, ROWS_PT)])

            @pl.when(s == 0)
            def _ot0():
                pltpu.sync_copy(acc_sp.at[pl.ds(TAILOFF, TAIL)],
                                outA.at[p, pl.ds(TAILOFF, TAIL)])

        @pl.when(c == 1)
        def _o1():
            pltpu.sync_copy(acc_sp.at[pl.ds(s * ROWS_PT, ROWS_PT)],
                            outB.at[p, pl.ds(s * ROWS_PT, ROWS_PT)])

            @pl.when(s == 0)
            def _ot1():
                pltpu.sync_copy(acc_sp.at[pl.ds(TAILOFF, TAIL)],
                                outB.at[p, pl.ds(TAILOFF, TAIL)])

        plsc.subcore_barrier()
        _zero_acc()
        plsc.subcore_barrier()
        return carry

    lax.fori_loop(0, NSP, subpass, 0)


_sc_call = functools.partial(
    pl.kernel,
    _sc_body,
    out_type=(jax.ShapeDtypeStruct((NSP, N, SW), jnp.float32),) * 2,
    mesh=plsc.VectorSubcoreMesh(core_axis_name="c", subcore_axis_name="s"),
    compiler_params=pltpu.CompilerParams(use_tc_tiling_on_sc=False,
                                         needs_layout_passes=False),
    scratch_types=(
        pltpu.VMEM((EPTP,), jnp.int32),
        pltpu.VMEM((EPTP,), jnp.int32),
        pltpu.VMEM((2, CA, 16), jnp.float32),
        pltpu.VMEM((2, CA, 16), jnp.float32),
        pltpu.VMEM((2, CA, 16), jnp.float32),
        pltpu.VMEM((2 * HC, CA), jnp.int32),
        pltpu.VMEM((2, CB), jnp.int32),
        pltpu.VMEM((2, CB), jnp.int32),
        pltpu.VMEM((2, CB, SW), jnp.float32),
        pltpu.VMEM((2, CB, SW), jnp.float32),
        pltpu.VMEM((HC * EPTP,), jnp.float32),
        pltpu.VMEM((DPAD,), jnp.float32),
        pltpu.VMEM_SHARED((N, SW), jnp.float32),
        pltpu.VMEM_SHARED((DPAD,), jnp.float32),
        pltpu.SemaphoreType.DMA((2,)),
        pltpu.SemaphoreType.DMA((2,)),
    ),
)()


def kernel(x, edge_index, edge_weight, W, a_src, a_dst, edge_proj_w,
           edge_proj_b, bias):
    src = edge_index[0]
    dst = edge_index[1]

    W2 = W.transpose(1, 0, 2).reshape(D, H * HD)
    hidx = jnp.arange(H * HD)
    As = jnp.zeros((H * HD, H), jnp.float32).at[hidx, hidx // HD].set(
        a_src[:, :, 0].reshape(-1))
    Ad = jnp.zeros((H * HD, H), jnp.float32).at[hidx, hidx // HD].set(
        a_dst[:, :, 0].reshape(-1))

    BLK = 1000
    hA, hB, asp, adp = pl.pallas_call(
        _proj_body,
        grid=(N // BLK,),
        in_specs=[
            pl.BlockSpec((BLK, D), lambda i: (i, 0)),
            pl.BlockSpec((D, H * HD), lambda i: (0, 0)),
            pl.BlockSpec((H * HD, H), lambda i: (0, 0)),
            pl.BlockSpec((H * HD, H), lambda i: (0, 0)),
        ],
        out_specs=[
            pl.BlockSpec((BLK, HC * HD), lambda i: (i, 0)),
            pl.BlockSpec((BLK, HC * HD), lambda i: (i, 0)),
            pl.BlockSpec((BLK, 16), lambda i: (i, 0)),
            pl.BlockSpec((BLK, 16), lambda i: (i, 0)),
        ],
        out_shape=[
            jax.ShapeDtypeStruct((N, HC * HD), jnp.float32),
            jax.ShapeDtypeStruct((N, HC * HD), jnp.float32),
            jax.ShapeDtypeStruct((N, 16), jnp.float32),
            jax.ShapeDtypeStruct((N, 16), jnp.float32),
        ],
    )(x, W2, As, Ad)

    pw_p = jnp.concatenate([edge_proj_w.reshape(1, H),
                            jnp.zeros((1, 16 - H), jnp.float32)], axis=1)
    pb_p = jnp.concatenate([edge_proj_b.reshape(1, H),
                            jnp.zeros((1, 16 - H), jnp.float32)], axis=1)
    infl_p = pl.pallas_call(
        _infl_body,
        grid=(NS,),
        in_specs=[
            pl.BlockSpec((EPT, 1), lambda i: (i, 0)),
            pl.BlockSpec((1, 16), lambda i: (0, 0)),
            pl.BlockSpec((1, 16), lambda i: (0, 0)),
        ],
        out_specs=pl.BlockSpec((EPTP, 16), lambda i: (i, 0)),
        out_shape=jax.ShapeDtypeStruct((EP, 16), jnp.float32),
    )(edge_weight.reshape(E, 1), pw_p, pb_p)

    padn = EPTP - EPT
    src_p = jnp.concatenate(
        [src.reshape(NS, EPT), jnp.zeros((NS, padn), jnp.int32)],
        axis=1).reshape(-1)
    dst_p = jnp.concatenate(
        [dst.reshape(NS, EPT), jnp.zeros((NS, padn), jnp.int32)],
        axis=1).reshape(-1)

    hA = hA.reshape(N * NSP, SW)
    hB = hB.reshape(N * NSP, SW)
    zrows = jnp.zeros((ROWS_PT, SW), jnp.float32)
    zflat = jnp.zeros((DLEN,), jnp.float32)
    oA, oB = _sc_call(hA, hB, asp, adp, infl_p, src_p, dst_p, zrows, zflat)
    out = jnp.concatenate(
        [oA.transpose(1, 0, 2).reshape(N, HC * HD),
         oB.transpose(1, 0, 2).reshape(N, HC * HD)], axis=1)
    return out + bias
